# baseline (speedup 1.0000x reference)
import jax
import jax.numpy as jnp
from jax.experimental import pallas as pl

S = 2048
D = 1024
H = 16
HD = D // H
K_KEEP = S // 2
ROWS = 1024
BLK = 512
NBITS = 16


def _qkv_body(x_ref, wq_ref, wk_ref, wv_ref, b_ref, q_ref, k_ref, v_ref):
    x = x_ref[...]
    nt = (((1,), (1,)), ((), ()))
    for w_ref, bi, o_ref in ((wq_ref, 0, q_ref), (wk_ref, 1, k_ref),
                             (wv_ref, 2, v_ref)):
        y = jax.lax.dot_general(x, w_ref[...], nt,
                                preferred_element_type=jnp.float32)
        y = y + b_ref[bi:bi + 1, :]
        for h in range(H):
            o_ref[h] = y[:, h * HD:(h + 1) * HD]


def _key_to_float(cand):
    mask7f = jnp.int32(0x7FFFFFFF)
    u = jnp.where(cand < 0, jnp.bitwise_and(cand, mask7f),
                  jnp.bitwise_not(cand))
    return jax.lax.bitcast_convert_type(u, jnp.float32)


def _attn_body(q_ref, k_ref, v_ref, o_ref):
    q = q_ref[0] * jnp.float32(0.125)
    k = k_ref[0]
    s = jax.lax.dot_general(q, k, (((1,), (1,)), ((), ())),
                            preferred_element_type=jnp.float32)

    kf = jnp.float32(K_KEEP)
    sb = s.astype(jnp.bfloat16)
    one_b = jnp.bfloat16(1.0)
    zero_b = jnp.bfloat16(0.0)

    def count_ge_b(tb):
        selb = jnp.where(sb >= tb, one_b, zero_b)
        acc = selb[:, 0:128]
        for j in range(1, 16):
            acc = acc + selb[:, j * 128:(j + 1) * 128]
        return jnp.sum(acc.astype(jnp.float32), axis=1, keepdims=True)

    def step(i, t):
        bit = jnp.left_shift(jnp.int32(1), 31 - i)
        cand = jnp.bitwise_or(t, bit)
        tb = _key_to_float(cand).astype(jnp.bfloat16)
        cnt = count_ge_b(tb)
        return jnp.where(cnt >= kf, cand, t)

    t = jnp.zeros((ROWS, 1), jnp.int32)
    for i in range(NBITS):
        t = step(i, t)
    thr_b = _key_to_float(t).astype(jnp.bfloat16)

    m = jnp.max(s, axis=1, keepdims=True)
    p = jnp.where(sb >= thr_b, jnp.exp(s - m), jnp.float32(0.0))
    v_aug = jnp.concatenate([v_ref[0], jnp.ones((S, HD), jnp.float32)],
                            axis=1)
    cd = jax.lax.dot_general(p, v_aug, (((1,), (0,)), ((), ())),
                             preferred_element_type=jnp.float32)
    o_ref[0] = cd[:, 0:HD] / cd[:, HD:HD + 1]


def _proj_body(c_ref, wo_ref, bo_ref, o_ref):
    acc = jnp.broadcast_to(bo_ref[...], (BLK, D))
    for h in range(H):
        acc = acc + jnp.dot(c_ref[h], wo_ref[h],
                            preferred_element_type=jnp.float32)
    o_ref[...] = acc


@jax.jit
def kernel(hidden_states, Wq, bq, Wk, bk, Wv, bv, Wo, bo):
    x = hidden_states.reshape(S, D)
    wo_t = Wo.T.reshape(H, HD, D)
    b_qkv = jnp.stack([bq, bk, bv])

    q, k, v = pl.pallas_call(
        _qkv_body,
        grid=(S // BLK,),
        in_specs=[
            pl.BlockSpec((BLK, D), lambda r: (r, 0)),
            pl.BlockSpec((D, D), lambda r: (0, 0)),
            pl.BlockSpec((D, D), lambda r: (0, 0)),
            pl.BlockSpec((D, D), lambda r: (0, 0)),
            pl.BlockSpec((3, D), lambda r: (0, 0)),
        ],
        out_specs=[
            pl.BlockSpec((H, BLK, HD), lambda r: (0, r, 0)),
            pl.BlockSpec((H, BLK, HD), lambda r: (0, r, 0)),
            pl.BlockSpec((H, BLK, HD), lambda r: (0, r, 0)),
        ],
        out_shape=[jax.ShapeDtypeStruct((H, S, HD), jnp.float32)] * 3,
    )(x, Wq, Wk, Wv, b_qkv)

    ctx = pl.pallas_call(
        _attn_body,
        grid=(H, S // ROWS),
        in_specs=[
            pl.BlockSpec((1, ROWS, HD), lambda h, r: (h, r, 0)),
            pl.BlockSpec((1, S, HD), lambda h, r: (h, 0, 0)),
            pl.BlockSpec((1, S, HD), lambda h, r: (h, 0, 0)),
        ],
        out_specs=pl.BlockSpec((1, ROWS, HD), lambda h, r: (h, r, 0)),
        out_shape=jax.ShapeDtypeStruct((H, S, HD), jnp.float32),
    )(q, k, v)

    out = pl.pallas_call(
        _proj_body,
        grid=(S // BLK,),
        in_specs=[
            pl.BlockSpec((H, BLK, HD), lambda r: (0, r, 0)),
            pl.BlockSpec((H, HD, D), lambda r: (0, 0, 0)),
            pl.BlockSpec((1, D), lambda r: (0, 0)),
        ],
        out_specs=pl.BlockSpec((BLK, D), lambda r: (r, 0)),
        out_shape=jax.ShapeDtypeStruct((S, D), jnp.float32),
    )(ctx, wo_t, bo.reshape(1, D))

    return out.reshape(1, S, D)


# speedup vs baseline: 1.0720x; 1.0720x over previous
import jax
import jax.numpy as jnp
from jax.experimental import pallas as pl

S = 2048
D = 1024
H = 16
HD = D // H
K_KEEP = S // 2
ROWS = 2048
BLK = 512
NBITS = 16


def _qkv_body(x_ref, wq_ref, wk_ref, wv_ref, b_ref, q_ref, k_ref, v_ref):
    x = x_ref[...]
    nt = (((1,), (1,)), ((), ()))
    for w_ref, bi, o_ref in ((wq_ref, 0, q_ref), (wk_ref, 1, k_ref),
                             (wv_ref, 2, v_ref)):
        y = jax.lax.dot_general(x, w_ref[...], nt,
                                preferred_element_type=jnp.float32)
        y = y + b_ref[bi:bi + 1, :]
        for h in range(H):
            o_ref[h] = y[:, h * HD:(h + 1) * HD]


def _key_to_float(cand):
    mask7f = jnp.int32(0x7FFFFFFF)
    u = jnp.where(cand < 0, jnp.bitwise_and(cand, mask7f),
                  jnp.bitwise_not(cand))
    return jax.lax.bitcast_convert_type(u, jnp.float32)


def _attn_body(q_ref, k_ref, v_ref, o_ref):
    q = q_ref[0] * jnp.float32(0.125)
    k = k_ref[0]
    s = jax.lax.dot_general(q, k, (((1,), (1,)), ((), ())),
                            preferred_element_type=jnp.float32)

    kf = jnp.float32(K_KEEP)
    sb = s.astype(jnp.bfloat16)
    one_b = jnp.bfloat16(1.0)
    zero_b = jnp.bfloat16(0.0)

    def count_ge_b(tb):
        selb = jnp.where(sb >= tb, one_b, zero_b)
        acc = selb[:, 0:128]
        for j in range(1, 16):
            acc = acc + selb[:, j * 128:(j + 1) * 128]
        return jnp.sum(acc.astype(jnp.float32), axis=1, keepdims=True)

    def step(i, t):
        bit = jnp.left_shift(jnp.int32(1), 31 - i)
        cand = jnp.bitwise_or(t, bit)
        tb = _key_to_float(cand).astype(jnp.bfloat16)
        cnt = count_ge_b(tb)
        return jnp.where(cnt >= kf, cand, t)

    t = jnp.zeros((ROWS, 1), jnp.int32)
    for i in range(NBITS):
        t = step(i, t)
    thr_b = _key_to_float(t).astype(jnp.bfloat16)

    m = jnp.max(s, axis=1, keepdims=True)
    p = jnp.where(sb >= thr_b, jnp.exp(s - m), jnp.float32(0.0))
    ones_cnt = jnp.ones((S, 8), jnp.float32)
    denom = jax.lax.dot_general(p, ones_cnt, (((1,), (0,)), ((), ())),
                                preferred_element_type=jnp.float32)[:, 0:1]
    ctx = jax.lax.dot_general(p, v_ref[0], (((1,), (0,)), ((), ())),
                              preferred_element_type=jnp.float32)
    o_ref[0] = ctx / denom


def _proj_body(c_ref, wo_ref, bo_ref, o_ref):
    acc = jnp.broadcast_to(bo_ref[...], (BLK, D))
    for h in range(H):
        acc = acc + jnp.dot(c_ref[h], wo_ref[h],
                            preferred_element_type=jnp.float32)
    o_ref[...] = acc


@jax.jit
def kernel(hidden_states, Wq, bq, Wk, bk, Wv, bv, Wo, bo):
    x = hidden_states.reshape(S, D)
    wo_t = Wo.T.reshape(H, HD, D)
    b_qkv = jnp.stack([bq, bk, bv])

    q, k, v = pl.pallas_call(
        _qkv_body,
        grid=(S // BLK,),
        in_specs=[
            pl.BlockSpec((BLK, D), lambda r: (r, 0)),
            pl.BlockSpec((D, D), lambda r: (0, 0)),
            pl.BlockSpec((D, D), lambda r: (0, 0)),
            pl.BlockSpec((D, D), lambda r: (0, 0)),
            pl.BlockSpec((3, D), lambda r: (0, 0)),
        ],
        out_specs=[
            pl.BlockSpec((H, BLK, HD), lambda r: (0, r, 0)),
            pl.BlockSpec((H, BLK, HD), lambda r: (0, r, 0)),
            pl.BlockSpec((H, BLK, HD), lambda r: (0, r, 0)),
        ],
        out_shape=[jax.ShapeDtypeStruct((H, S, HD), jnp.float32)] * 3,
    )(x, Wq, Wk, Wv, b_qkv)

    ctx = pl.pallas_call(
        _attn_body,
        grid=(H, S // ROWS),
        in_specs=[
            pl.BlockSpec((1, ROWS, HD), lambda h, r: (h, r, 0)),
            pl.BlockSpec((1, S, HD), lambda h, r: (h, 0, 0)),
            pl.BlockSpec((1, S, HD), lambda h, r: (h, 0, 0)),
        ],
        out_specs=pl.BlockSpec((1, ROWS, HD), lambda h, r: (h, r, 0)),
        out_shape=jax.ShapeDtypeStruct((H, S, HD), jnp.float32),
    )(q, k, v)

    out = pl.pallas_call(
        _proj_body,
        grid=(S // BLK,),
        in_specs=[
            pl.BlockSpec((H, BLK, HD), lambda r: (0, r, 0)),
            pl.BlockSpec((H, HD, D), lambda r: (0, 0, 0)),
            pl.BlockSpec((1, D), lambda r: (0, 0)),
        ],
        out_specs=pl.BlockSpec((BLK, D), lambda r: (r, 0)),
        out_shape=jax.ShapeDtypeStruct((S, D), jnp.float32),
    )(ctx, wo_t, bo.reshape(1, D))

    return out.reshape(1, S, D)
